# baseline (speedup 1.0000x reference)
import functools

import jax
import jax.numpy as jnp
from jax import lax
from jax.experimental import pallas as pl
from jax.experimental.pallas import tpu as pltpu
from jax.experimental.pallas import tpu_sc as plsc

B = 16384
EMB = 32
NC = 2
NS = 16
NW = NC * NS
BPW = B // NW
CH = 8
NPAIR = BPW // (2 * CH)

BLK = 2048
NB = B // BLK

_f32 = jnp.float32
_i32 = jnp.int32


_sc_mesh = plsc.VectorSubcoreMesh(core_axis_name="c", subcore_axis_name="s")


@functools.partial(
    pl.kernel,
    mesh=_sc_mesh,
    compiler_params=pltpu.CompilerParams(use_tc_tiling_on_sc=True,
                                         needs_layout_passes=False),
    out_type=[
        jax.ShapeDtypeStruct((EMB, B), _f32),
    ],
    scratch_types=[
        pltpu.VMEM((B,), _i32),
        pltpu.VMEM((2 * CH, EMB, 128), _f32),
        pltpu.VMEM((EMB, BPW), _f32),
        pltpu.SemaphoreType.DMA,
        pltpu.SemaphoreType.DMA,
    ],
)
def _sc_user_gather(uid_hbm, utabT_hbm, xuT_hbm, uidx_v, win_v, rowsT_v,
                    sem_a, sem_b):
    wid = lax.axis_index("s") * NC + lax.axis_index("c")
    base = wid * BPW
    pltpu.sync_copy(uid_hbm, uidx_v)

    iota0 = lax.iota(_i32, 16)
    iota1 = iota0 + 16

    def fire(ids16, lo, bufbase, sem):
        for j in range(CH):
            rid = ids16[lo + j]
            col = pl.multiple_of((rid // 128) * 128, 128)
            pltpu.async_copy(
                utabT_hbm.at[:, pl.ds(col, 128)],
                win_v.at[bufbase + j], sem)

    def drain(sem):
        for j in range(CH):
            pltpu.make_async_copy(
                utabT_hbm.at[:, pl.ds(0, 128)],
                win_v.at[j], sem).wait()

    def process(ids16, lo, kpair, bufbase):
        for j in range(CH):
            rid = ids16[lo + j]
            lane = lax.broadcast(lax.rem(rid, 128), (16,))
            spos = lax.broadcast(kpair * 2 * CH + lo + j, (16,))
            v0 = plsc.load_gather(win_v.at[bufbase + j], [iota0, lane])
            v1 = plsc.load_gather(win_v.at[bufbase + j], [iota1, lane])
            plsc.store_scatter(rowsT_v, [iota0, spos], v0)
            plsc.store_scatter(rowsT_v, [iota1, spos], v1)

    ids_first = uidx_v[pl.ds(base, 16)]
    fire(ids_first, 0, 0, sem_a)

    def pair(k, carry):
        ids16 = uidx_v[pl.ds(base + k * 2 * CH, 16)]
        fire(ids16, CH, CH, sem_b)
        drain(sem_a)
        process(ids16, 0, k, 0)

        @pl.when(k < NPAIR - 1)
        def _():
            ids_next = uidx_v[pl.ds(base + (k + 1) * 2 * CH, 16)]
            fire(ids_next, 0, 0, sem_a)

        drain(sem_b)
        process(ids16, CH, k, CH)
        return carry

    lax.fori_loop(0, NPAIR, pair, 0)
    pltpu.sync_copy(rowsT_v,
                    xuT_hbm.at[:, pl.ds(pl.multiple_of(base, 128), BPW)])


@functools.partial(
    pl.kernel,
    mesh=_sc_mesh,
    compiler_params=pltpu.CompilerParams(use_tc_tiling_on_sc=True,
                                         needs_layout_passes=False),
    out_type=[
        jax.ShapeDtypeStruct((B, 128), _f32),
    ],
    scratch_types=[
        pltpu.VMEM((B,), _i32),
        pltpu.VMEM((BPW,), _i32),
        pltpu.VMEM((BPW, 128), _f32),
        pltpu.SemaphoreType.DMA,
    ],
)
def _sc_movie_gather(mid_hbm, mtabp_hbm, xm_hbm, midx_v, mlist_v, mrows_v,
                     sem_m):
    wid = lax.axis_index("s") * NC + lax.axis_index("c")
    base = wid * BPW
    pltpu.sync_copy(mid_hbm, midx_v)
    for t in range(BPW // 16):
        mlist_v[pl.ds(16 * t, 16)] = midx_v[pl.ds(base + 16 * t, 16)]
    pltpu.async_copy(mtabp_hbm.at[mlist_v], mrows_v, sem_m).wait()
    pltpu.sync_copy(mrows_v,
                    xm_hbm.at[pl.ds(pl.multiple_of(base, 128), BPW)])


def _mlp_body(xuT_ref, xm_ref, ex_ref,
              wpack_ref,
              w1u_ref, wage_ref, wyear_ref,
              b1_ref, gamma_ref, beta_ref,
              w2_ref, b2_ref, w3_ref, b3_ref, w4_ref, b4_ref,
              out_ref, h_scr, acc_s, acc_q, c_scr):
    p = pl.program_id(0)
    i = pl.program_id(1)
    dot = functools.partial(jnp.dot, preferred_element_type=_f32,
                            precision=lax.Precision.HIGHEST)

    @pl.when(p == 0)
    def _phase0():
        ex = ex_ref[...]
        age_s = (ex[:, 0:1] - 30.0) * 0.05
        year_s = (ex[:, 1:2] - 2000.0) * 0.05
        gen_oh = (ex[:, 2:3].astype(_i32)
                  == lax.broadcasted_iota(_i32, (1, 8), 1)
                  ).astype(_f32)
        genre_oh = (ex[:, 3:4].astype(_i32)
                    == lax.broadcasted_iota(_i32, (1, 32), 1)
                    ).astype(_f32)
        packed = jnp.concatenate([xm_ref[:, 0:32], gen_oh, genre_oh],
                                 axis=1)
        h = (lax.dot_general(xuT_ref[...], w1u_ref[...],
                             (((0,), (0,)), ((), ())),
                             preferred_element_type=_f32,
                             precision=lax.Precision.HIGHEST)
             + dot(packed, wpack_ref[...])
             + age_s * wage_ref[...]
             + year_s * wyear_ref[...]
             + b1_ref[...])
        h_scr[pl.ds(i * BLK, BLK), :] = h

        @pl.when(i == 0)
        def _init():
            c_scr[...] = h[0:1, :]
            acc_s[...] = jnp.zeros_like(acc_s)
            acc_q[...] = jnp.zeros_like(acc_q)

        acc_s[...] += jnp.sum(h, axis=0, keepdims=True)
        d = h - c_scr[...]
        acc_q[...] += jnp.sum(d * d, axis=0, keepdims=True)

    @pl.when(p == 1)
    def _phase1():
        mu = acc_s[...] * (1.0 / B)
        mc = mu - c_scr[...]
        var = acc_q[...] * (1.0 / B) - mc * mc
        scale = lax.rsqrt(var + 1e-5) * gamma_ref[...]
        h = h_scr[pl.ds(i * BLK, BLK), :]
        h = jnp.maximum((h - mu) * scale + beta_ref[...], 0.0)
        h = jnp.maximum(dot(h, w2_ref[...]) + b2_ref[...], 0.0)
        h = jnp.maximum(dot(h, w3_ref[...]) + b3_ref[...], 0.0)
        logit = dot(h, w4_ref[...]) + b4_ref[...]
        out_ref[...] = jax.nn.sigmoid(logit) * 10.0


def _full(shape):
    return pl.BlockSpec(shape, lambda p, i: (0, 0))


_mlp_call = pl.pallas_call(
    _mlp_body,
    grid=(2, NB),
    in_specs=[
        pl.BlockSpec((EMB, BLK), lambda p, i: (0, i * (1 - p))),
        pl.BlockSpec((BLK, 128), lambda p, i: (i * (1 - p), 0)),
        pl.BlockSpec((BLK, 4), lambda p, i: (i * (1 - p), 0)),
        _full((72, 128)),
        _full((EMB, 128)),
        _full((1, 128)),
        _full((1, 128)),
        _full((1, 128)),
        _full((1, 128)),
        _full((1, 128)),
        _full((128, 64)),
        _full((1, 64)),
        _full((64, 32)),
        _full((1, 32)),
        _full((32, 1)),
        _full((1, 1)),
    ],
    out_specs=pl.BlockSpec((BLK, 1), lambda p, i: (i, 0)),
    out_shape=jax.ShapeDtypeStruct((B, 1), _f32),
    scratch_shapes=[
        pltpu.VMEM((B, 128), _f32),
        pltpu.VMEM((1, 128), _f32),
        pltpu.VMEM((1, 128), _f32),
        pltpu.VMEM((1, 128), _f32),
    ],
)


def kernel(u_id, m_id, u_age, u_gender, m_year, m_genre,
           user_emb, movie_emb, gender_emb, genre_emb,
           W1, b1, gamma, beta, W2, b2, W3, b3, W4, b4):
    (xuT,) = _sc_user_gather(u_id, user_emb.T)
    mtab_pad = jnp.pad(movie_emb, ((0, 0), (0, 96)))
    (xm,) = _sc_movie_gather(m_id, mtab_pad)

    extras = jnp.stack(
        [u_age, m_year, u_gender.astype(_f32), m_genre.astype(_f32)], axis=1)
    gemb8 = jnp.zeros((8, 8), _f32).at[0:3, :].set(gender_emb)
    wpack = jnp.concatenate(
        [W1[:, 32:64].T, gemb8 @ W1[:, 64:72].T, genre_emb @ W1[:, 72:88].T],
        axis=0)

    return _mlp_call(
        xuT, xm, extras,
        wpack,
        W1[:, 0:32].T,
        W1[:, 88].reshape(1, 128), W1[:, 89].reshape(1, 128),
        b1.reshape(1, 128), gamma.reshape(1, 128), beta.reshape(1, 128),
        W2.T, b2.reshape(1, 64), W3.T, b3.reshape(1, 32),
        W4.T, b4.reshape(1, 1),
    )


# speedup vs baseline: 1.0030x; 1.0030x over previous
import functools

import jax
import jax.numpy as jnp
from jax import lax
from jax.experimental import pallas as pl
from jax.experimental.pallas import tpu as pltpu
from jax.experimental.pallas import tpu_sc as plsc

B = 16384
EMB = 32
NC = 2
NS = 16
NW = NC * NS
BPW = B // NW
CH = 8
NPAIR = BPW // (2 * CH)

BLK = 2048
NB = B // BLK

_f32 = jnp.float32
_i32 = jnp.int32


_sc_mesh = plsc.VectorSubcoreMesh(core_axis_name="c", subcore_axis_name="s")


@functools.partial(
    pl.kernel,
    mesh=_sc_mesh,
    compiler_params=pltpu.CompilerParams(use_tc_tiling_on_sc=True,
                                         needs_layout_passes=False),
    out_type=[
        jax.ShapeDtypeStruct((EMB, B), _f32),
    ],
    scratch_types=[
        pltpu.VMEM((B,), _i32),
        pltpu.VMEM((2 * CH, EMB, 128), _f32),
        pltpu.VMEM((EMB, BPW), _f32),
        pltpu.SemaphoreType.DMA,
        pltpu.SemaphoreType.DMA,
    ],
)
def _sc_user_gather(uid_hbm, utabT_hbm, xuT_hbm, uidx_v, win_v, rowsT_v,
                    sem_a, sem_b):
    wid = lax.axis_index("s") * NC + lax.axis_index("c")
    base = wid * BPW
    pltpu.sync_copy(uid_hbm, uidx_v)

    iota0 = lax.iota(_i32, 16)
    iota1 = iota0 + 16

    def fire(ids16, lo, bufbase, sem):
        for j in range(CH):
            rid = ids16[lo + j]
            col = pl.multiple_of((rid // 128) * 128, 128)
            pltpu.async_copy(
                utabT_hbm.at[:, pl.ds(col, 128)],
                win_v.at[bufbase + j], sem)

    def drain(sem):
        for j in range(CH):
            pltpu.make_async_copy(
                utabT_hbm.at[:, pl.ds(0, 128)],
                win_v.at[j], sem).wait()

    def process(ids16, lo, kpair, bufbase):
        for j in range(CH):
            rid = ids16[lo + j]
            lane = lax.broadcast(lax.rem(rid, 128), (16,))
            spos = lax.broadcast(kpair * 2 * CH + lo + j, (16,))
            v0 = plsc.load_gather(win_v.at[bufbase + j], [iota0, lane])
            v1 = plsc.load_gather(win_v.at[bufbase + j], [iota1, lane])
            plsc.store_scatter(rowsT_v, [iota0, spos], v0)
            plsc.store_scatter(rowsT_v, [iota1, spos], v1)

    ids_first = uidx_v[pl.ds(base, 16)]
    fire(ids_first, 0, 0, sem_a)

    def pair(k, carry):
        ids16 = uidx_v[pl.ds(base + k * 2 * CH, 16)]
        fire(ids16, CH, CH, sem_b)
        drain(sem_a)
        process(ids16, 0, k, 0)

        @pl.when(k < NPAIR - 1)
        def _():
            ids_next = uidx_v[pl.ds(base + (k + 1) * 2 * CH, 16)]
            fire(ids_next, 0, 0, sem_a)

        drain(sem_b)
        process(ids16, CH, k, CH)
        return carry

    lax.fori_loop(0, NPAIR, pair, 0)
    pltpu.sync_copy(rowsT_v,
                    xuT_hbm.at[:, pl.ds(pl.multiple_of(base, 128), BPW)])


@functools.partial(
    pl.kernel,
    mesh=_sc_mesh,
    compiler_params=pltpu.CompilerParams(use_tc_tiling_on_sc=True,
                                         needs_layout_passes=False),
    out_type=[
        jax.ShapeDtypeStruct((B, 128), _f32),
    ],
    scratch_types=[
        pltpu.VMEM((B,), _i32),
        pltpu.VMEM((BPW,), _i32),
        pltpu.VMEM((BPW, 128), _f32),
        pltpu.SemaphoreType.DMA,
    ],
)
def _sc_movie_gather(mid_hbm, mtabp_hbm, xm_hbm, midx_v, mlist_v, mrows_v,
                     sem_m):
    wid = lax.axis_index("s") * NC + lax.axis_index("c")
    base = wid * BPW
    pltpu.sync_copy(mid_hbm, midx_v)
    for t in range(BPW // 16):
        mlist_v[pl.ds(16 * t, 16)] = midx_v[pl.ds(base + 16 * t, 16)]
    pltpu.async_copy(mtabp_hbm.at[mlist_v], mrows_v, sem_m).wait()
    pltpu.sync_copy(mrows_v,
                    xm_hbm.at[pl.ds(pl.multiple_of(base, 128), BPW)])


def _mlp_body(xuT_ref, xm_ref, ex_ref,
              wpack_ref,
              w1u_ref, wage_ref, wyear_ref,
              b1_ref, gamma_ref, beta_ref,
              w2_ref, b2_ref, w3_ref, b3_ref, w4_ref, b4_ref,
              out_ref, h_scr, acc_s, acc_q, c_scr):
    p = pl.program_id(0)
    i = pl.program_id(1)
    dot = functools.partial(jnp.dot, preferred_element_type=_f32,
                            precision=lax.Precision.HIGHEST)

    @pl.when(p == 0)
    def _phase0():
        ex = ex_ref[...]
        age_s = (ex[:, 0:1] - 30.0) * 0.05
        year_s = (ex[:, 1:2] - 2000.0) * 0.05
        gen_oh = (ex[:, 2:3].astype(_i32)
                  == lax.broadcasted_iota(_i32, (1, 8), 1)
                  ).astype(_f32)
        genre_oh = (ex[:, 3:4].astype(_i32)
                    == lax.broadcasted_iota(_i32, (1, 32), 1)
                    ).astype(_f32)
        packed = jnp.concatenate([xm_ref[:, 0:32], gen_oh, genre_oh],
                                 axis=1)
        h = (lax.dot_general(xuT_ref[...], w1u_ref[...],
                             (((0,), (0,)), ((), ())),
                             preferred_element_type=_f32,
                             precision=lax.Precision.HIGHEST)
             + dot(packed, wpack_ref[...])
             + age_s * wage_ref[...]
             + year_s * wyear_ref[...]
             + b1_ref[...])
        h_scr[pl.ds(i * BLK, BLK), :] = h

        @pl.when(i == 0)
        def _init():
            c_scr[...] = h[0:1, :]
            acc_s[...] = jnp.zeros_like(acc_s)
            acc_q[...] = jnp.zeros_like(acc_q)

        acc_s[...] += jnp.sum(h, axis=0, keepdims=True)
        d = h - c_scr[...]
        acc_q[...] += jnp.sum(d * d, axis=0, keepdims=True)

    @pl.when(p == 1)
    def _phase1():
        mu = acc_s[...] * (1.0 / B)
        mc = mu - c_scr[...]
        var = acc_q[...] * (1.0 / B) - mc * mc
        scale = lax.rsqrt(var + 1e-5) * gamma_ref[...]
        h = h_scr[pl.ds(i * BLK, BLK), :]
        h = jnp.maximum((h - mu) * scale + beta_ref[...], 0.0)
        h = jnp.maximum(dot(h, w2_ref[...]) + b2_ref[...], 0.0)
        h = jnp.maximum(dot(h, w3_ref[...]) + b3_ref[...], 0.0)
        logit = dot(h, w4_ref[...]) + b4_ref[...]
        out_ref[...] = jax.nn.sigmoid(logit) * 10.0


def _full(shape):
    return pl.BlockSpec(shape, lambda p, i: (0, 0))


_mlp_call = pl.pallas_call(
    _mlp_body,
    grid=(2, NB),
    in_specs=[
        pl.BlockSpec((EMB, BLK), lambda p, i: (0, i)),
        pl.BlockSpec((BLK, 128), lambda p, i: (i, 0)),
        pl.BlockSpec((BLK, 4), lambda p, i: (i, 0)),
        _full((72, 128)),
        _full((EMB, 128)),
        _full((1, 128)),
        _full((1, 128)),
        _full((1, 128)),
        _full((1, 128)),
        _full((1, 128)),
        _full((128, 64)),
        _full((1, 64)),
        _full((64, 32)),
        _full((1, 32)),
        _full((32, 1)),
        _full((1, 1)),
    ],
    out_specs=pl.BlockSpec((BLK, 1), lambda p, i: (i, 0)),
    out_shape=jax.ShapeDtypeStruct((B, 1), _f32),
    scratch_shapes=[
        pltpu.VMEM((B, 128), _f32),
        pltpu.VMEM((1, 128), _f32),
        pltpu.VMEM((1, 128), _f32),
        pltpu.VMEM((1, 128), _f32),
    ],
)


def kernel(u_id, m_id, u_age, u_gender, m_year, m_genre,
           user_emb, movie_emb, gender_emb, genre_emb,
           W1, b1, gamma, beta, W2, b2, W3, b3, W4, b4):
    (xuT,) = _sc_user_gather(u_id, user_emb.T)
    mtab_pad = jnp.pad(movie_emb, ((0, 0), (0, 96)))
    (xm,) = _sc_movie_gather(m_id, mtab_pad)

    extras = jnp.stack(
        [u_age, m_year, u_gender.astype(_f32), m_genre.astype(_f32)], axis=1)
    gemb8 = jnp.zeros((8, 8), _f32).at[0:3, :].set(gender_emb)
    wpack = jnp.concatenate(
        [W1[:, 32:64].T, gemb8 @ W1[:, 64:72].T, genre_emb @ W1[:, 72:88].T],
        axis=0)

    return _mlp_call(
        xuT, xm, extras,
        wpack,
        W1[:, 0:32].T,
        W1[:, 88].reshape(1, 128), W1[:, 89].reshape(1, 128),
        b1.reshape(1, 128), gamma.reshape(1, 128), beta.reshape(1, 128),
        W2.T, b2.reshape(1, 64), W3.T, b3.reshape(1, 32),
        W4.T, b4.reshape(1, 1),
    )
